# baseline (speedup 1.0000x reference)
import functools

import jax
import jax.numpy as jnp
from jax import lax
from jax.experimental import pallas as pl
from jax.experimental.pallas import tpu as pltpu
from jax.experimental.pallas import tpu_sc as plsc

_NC = 2
_NS = 16
_NW = _NC * _NS
_L = 16


def _ceil_to(x, m):
    return (x + m - 1) // m * m


def _matmul_block(x_ref, w_ref, o_ref):
    o_ref[...] = jnp.dot(x_ref[...], w_ref[...],
                         preferred_element_type=jnp.float32)


def _shuffle(x, idx):
    dnums = lax.GatherDimensionNumbers(
        offset_dims=(), collapsed_slice_dims=(0,), start_index_map=(0,))
    return lax.gather(x, idx[:, None], dnums, (1,),
                      mode=lax.GatherScatterMode.PROMISE_IN_BOUNDS)


def _bf_sum(x):
    lane = lax.iota(jnp.int32, _L)
    for sh in (1, 2, 4, 8):
        x = x + _shuffle(x, lane ^ sh)
    return x


def _bf_max(x):
    lane = lax.iota(jnp.int32, _L)
    for sh in (1, 2, 4, 8):
        x = jnp.maximum(x, _shuffle(x, lane ^ sh))
    return x


def _make_edge_kernel(d, ew):
    mesh = plsc.VectorSubcoreMesh(core_axis_name="c", subcore_axis_name="s")
    grp = ew // _L

    npair = grp // 2
    tail = grp - npair * 2

    @functools.partial(
        pl.kernel,
        mesh=mesh,
        out_type=jax.ShapeDtypeStruct((ew * _NW,), jnp.float32),
        scratch_types=[
            pltpu.VMEM((ew,), jnp.int32),
            pltpu.VMEM((ew,), jnp.int32),
            pltpu.VMEM((_L, d), jnp.float32),
            pltpu.VMEM((_L, d), jnp.float32),
            pltpu.VMEM((_L, d), jnp.float32),
            pltpu.VMEM((_L, d), jnp.float32),
            pltpu.VMEM((ew,), jnp.float32),
            pltpu.VMEM((d,), jnp.float32),
            pltpu.SemaphoreType.DMA,
            pltpu.SemaphoreType.DMA,
            pltpu.SemaphoreType.DMA,
            pltpu.SemaphoreType.DMA,
        ],
        compiler_params=pltpu.CompilerParams(needs_layout_passes=False),
    )
    def edge_kernel(h_hbm, src_hbm, dst_hbm, avec_hbm, v_hbm,
                    srcv, dstv, hs0, hd0, hs1, hd1, vout, abuf,
                    sem1a, sem2a, sem1b, sem2b):
        wid = lax.axis_index("s") * _NC + lax.axis_index("c")
        base = wid * ew
        pltpu.sync_copy(src_hbm.at[pl.ds(base, ew)], srcv)
        pltpu.sync_copy(dst_hbm.at[pl.ds(base, ew)], dstv)
        pltpu.sync_copy(avec_hbm, abuf)
        lane = lax.iota(jnp.int32, _L)

        def compute(hs, hd, cs):
            vvec = jnp.zeros((_L,), jnp.float32)
            for e_i in range(_L):
                t = jnp.zeros((_L,), jnp.float32)
                for q in range(d // _L):
                    hsq = hs[e_i, pl.ds(q * _L, _L)]
                    hdq = hd[e_i, pl.ds(q * _L, _L)]
                    t = t + abuf[pl.ds(q * _L, _L)] * jnp.abs(hsq - hdq)
                vvec = vvec + jnp.where(lane == e_i, _bf_sum(t), 0.0)
            vout[pl.ds(cs, _L)] = jnp.maximum(vvec, 0.0)

        def pair(p, carry):
            cs0 = (2 * p) * _L
            cs1 = cs0 + _L
            cp1a = pltpu.async_copy(
                h_hbm.at[srcv.at[pl.ds(cs0, _L)]], hs0, sem1a)
            cp2a = pltpu.async_copy(
                h_hbm.at[dstv.at[pl.ds(cs0, _L)]], hd0, sem2a)
            cp1b = pltpu.async_copy(
                h_hbm.at[srcv.at[pl.ds(cs1, _L)]], hs1, sem1b)
            cp2b = pltpu.async_copy(
                h_hbm.at[dstv.at[pl.ds(cs1, _L)]], hd1, sem2b)
            cp1a.wait()
            cp2a.wait()
            compute(hs0, hd0, cs0)
            cp1b.wait()
            cp2b.wait()
            compute(hs1, hd1, cs1)
            return carry

        lax.fori_loop(0, npair, pair, 0)
        if tail:
            cs = (grp - 1) * _L
            cp1 = pltpu.async_copy(
                h_hbm.at[srcv.at[pl.ds(cs, _L)]], hs0, sem1a)
            cp2 = pltpu.async_copy(
                h_hbm.at[dstv.at[pl.ds(cs, _L)]], hd0, sem2a)
            cp1.wait()
            cp2.wait()
            compute(hs0, hd0, cs)
        pltpu.sync_copy(vout, v_hbm.at[pl.ds(base, ew)])

    return edge_kernel


_CAP = 64
_CAPC = _CAP // _L


def _make_row_kernel(n, rw, rw_pad):
    mesh = plsc.VectorSubcoreMesh(core_axis_name="c", subcore_axis_name="s")
    nq = n // _L
    nf = jnp.float32(n)

    @functools.partial(
        pl.kernel,
        mesh=mesh,
        out_type=jax.ShapeDtypeStruct((n, n), jnp.float32),
        scratch_types=[
            pltpu.VMEM((n,), jnp.float32),
            pltpu.VMEM((n,), jnp.float32),
            pltpu.VMEM((n,), jnp.float32),
            pltpu.VMEM((_CAP,), jnp.int32),
            pltpu.VMEM((_CAP,), jnp.float32),
            pltpu.VMEM((rw_pad,), jnp.int32),
            pltpu.SemaphoreType.DMA,
        ],
        compiler_params=pltpu.CompilerParams(needs_layout_passes=False),
    )
    def row_kernel(dst_hbm, v_hbm, rstart_hbm, out_hbm,
                   abuf, cbuf, outbuf, dstrow, vrow, rsv, semo):
        wid = lax.axis_index("s") * _NC + lax.axis_index("c")
        r0 = wid * rw
        nrows = jnp.clip(n - r0, 0, rw)
        pltpu.sync_copy(rstart_hbm.at[pl.ds(r0, rw_pad)], rsv)
        lane = lax.iota(jnp.int32, _L)
        zero16 = jnp.zeros((_L,), jnp.float32)
        ones16 = jnp.ones((_L,), jnp.float32)

        for q in range(nq):
            abuf[pl.ds(q * _L, _L)] = zero16
            cbuf[pl.ds(q * _L, _L)] = zero16

        def per_row(i, carry):
            row = r0 + i

            if True:
                rs = rsv[pl.ds(i, _L)][0]
                re = rsv[pl.ds(i + 1, _L)][0]
                c0 = rs // _L
                nch = (re + _L - 1) // _L - c0
                e0 = c0 * _L
                nseg = (nch + _CAPC - 1) // _CAPC

                def seg_bounds(s):
                    segbase = e0 + s * _CAP
                    jn = jnp.minimum(_CAPC, nch - s * _CAPC)
                    return segbase, jn

                def load_seg(segbase, with_v):
                    pltpu.sync_copy(dst_hbm.at[pl.ds(segbase, _CAP)], dstrow)
                    if with_v:
                        pltpu.sync_copy(v_hbm.at[pl.ds(segbase, _CAP)], vrow)

                def chunk_ctx(segbase, j):
                    dstc = dstrow[pl.ds(j * _L, _L)]
                    g = segbase + j * _L + lane
                    msk = (g >= rs) & (g < re)
                    return dstc, msk

                def p1_seg(s, c):
                    segbase, jn = seg_bounds(s)
                    load_seg(segbase, True)

                    def p1_j(j, cc):
                        dstc, msk = chunk_ctx(segbase, j)
                        vc = vrow[pl.ds(j * _L, _L)]
                        plsc.addupdate_scatter(abuf, [dstc], vc, mask=msk)
                        plsc.addupdate_scatter(cbuf, [dstc], ones16, mask=msk)
                        return cc
                    lax.fori_loop(0, jn, p1_j, 0)
                    return c
                lax.fori_loop(0, nseg, p1_seg, 0)

                def p2_seg(s, mv):
                    segbase, jn = seg_bounds(s)

                    @pl.when(nseg > 1)
                    def _():
                        load_seg(segbase, False)

                    def p2_j(j, mvj):
                        dstc, msk = chunk_ctx(segbase, j)
                        aj = plsc.load_gather(abuf, [dstc])
                        return jnp.maximum(mvj, jnp.where(msk, aj, 0.0))
                    return lax.fori_loop(0, jn, p2_j, mv)
                mvec = lax.fori_loop(0, nseg, p2_seg, zero16)
                msp = _bf_max(mvec)
                em = jnp.exp(-msp)

                def p3_seg(s, zv):
                    segbase, jn = seg_bounds(s)

                    @pl.when(nseg > 1)
                    def _():
                        load_seg(segbase, False)

                    def p3_j(j, zvj):
                        dstc, msk = chunk_ctx(segbase, j)
                        aj = plsc.load_gather(abuf, [dstc])
                        cj = plsc.load_gather(cbuf, [dstc])
                        t = (jnp.exp(aj - msp) - em) / cj
                        return zvj + jnp.where(msk, t, 0.0)
                    return lax.fori_loop(0, jn, p3_j, zv)
                zacc = lax.fori_loop(0, nseg, p3_seg, zero16)
                zsp = nf * em + _bf_sum(zacc)
                rzsp = 1.0 / zsp
                bsp = em * rzsp

                @pl.when(i > 0)
                def _():
                    pltpu.make_async_copy(
                        outbuf, out_hbm.at[row - 1], semo).wait()

                for q in range(nq):
                    outbuf[pl.ds(q * _L, _L)] = bsp

                def p4_seg(s, c):
                    segbase, jn = seg_bounds(s)

                    @pl.when(nseg > 1)
                    def _():
                        load_seg(segbase, False)

                    def p4_j(j, cc):
                        dstc, msk = chunk_ctx(segbase, j)
                        aj = plsc.load_gather(abuf, [dstc])
                        cj = plsc.load_gather(cbuf, [dstc])
                        w = jnp.exp(aj - msp) * rzsp + cj
                        plsc.store_scatter(outbuf, [dstc], w, mask=msk)
                        return cc
                    lax.fori_loop(0, jn, p4_j, 0)
                    return c
                lax.fori_loop(0, nseg, p4_seg, 0)

                pltpu.async_copy(outbuf, out_hbm.at[row], semo)

                def p5_seg(s, c):
                    segbase, jn = seg_bounds(s)

                    @pl.when(nseg > 1)
                    def _():
                        load_seg(segbase, False)

                    def p5_j(j, cc):
                        dstc, msk = chunk_ctx(segbase, j)
                        plsc.store_scatter(abuf, [dstc], zero16, mask=msk)
                        plsc.store_scatter(cbuf, [dstc], zero16, mask=msk)
                        return cc
                    lax.fori_loop(0, jn, p5_j, 0)
                    return c
                lax.fori_loop(0, nseg, p5_seg, 0)

            return carry

        lax.fori_loop(0, nrows, per_row, 0)

        @pl.when(nrows > 0)
        def _():
            pltpu.make_async_copy(
                outbuf, out_hbm.at[r0 + nrows - 1], semo).wait()

    return row_kernel


def kernel(inputs, edge, W, a):
    n, d_in = inputs.shape
    d_out = W.shape[1]
    e = edge.shape[1]

    src = edge[0].astype(jnp.int32)
    dst = edge[1].astype(jnp.int32)

    ew = _ceil_to(e, _NW * _L) // _NW
    e_pad = ew * _NW
    pad = e_pad - e
    if pad:
        src = jnp.concatenate([src, jnp.full((pad,), n, jnp.int32)])
        dst = jnp.concatenate([dst, jnp.zeros((pad,), jnp.int32)])
    src_g = jnp.minimum(src, n - 1)

    rw = _ceil_to(_ceil_to(n, _NW) // _NW, 8)
    rw_pad = rw + _L
    gl = _NW * rw + _L

    bm = 2000
    h = pl.pallas_call(
        _matmul_block,
        grid=(n // bm,),
        in_specs=[pl.BlockSpec((bm, d_in), lambda i: (i, 0)),
                  pl.BlockSpec((d_in, d_out), lambda i: (0, 0))],
        out_specs=pl.BlockSpec((bm, d_out), lambda i: (i, 0)),
        out_shape=jax.ShapeDtypeStruct((n, d_out), jnp.float32),
    )(inputs, W)

    avec = a.astype(jnp.float32).reshape(d_out)
    v = _make_edge_kernel(d_out, ew)(h, src_g, dst, avec)

    iot = jnp.arange(e_pad, dtype=jnp.int32)
    src_s, order = lax.sort([src, iot], num_keys=1, is_stable=False)
    row_start = jnp.searchsorted(
        src_s, jnp.minimum(jnp.arange(gl), n)).astype(jnp.int32)
    dst_s = dst[order]
    v_s = v[order]

    dst3 = jnp.concatenate([dst_s, jnp.zeros((_CAP,), jnp.int32)])
    v3 = jnp.concatenate([v_s, jnp.zeros((_CAP,), jnp.float32)])
    sgraph = _make_row_kernel(n, rw, rw_pad)(dst3, v3, row_start)

    return (h, sgraph)


# speedup vs baseline: 1.3700x; 1.3700x over previous
import functools

import jax
import jax.numpy as jnp
from jax import lax
from jax.experimental import pallas as pl
from jax.experimental.pallas import tpu as pltpu
from jax.experimental.pallas import tpu_sc as plsc

_NC = 2
_NS = 16
_NW = _NC * _NS
_L = 16


def _ceil_to(x, m):
    return (x + m - 1) // m * m


def _matmul_block(x_ref, w_ref, o_ref):
    o_ref[...] = jnp.dot(x_ref[...], w_ref[...],
                         preferred_element_type=jnp.float32)


def _shuffle(x, idx):
    dnums = lax.GatherDimensionNumbers(
        offset_dims=(), collapsed_slice_dims=(0,), start_index_map=(0,))
    return lax.gather(x, idx[:, None], dnums, (1,),
                      mode=lax.GatherScatterMode.PROMISE_IN_BOUNDS)


def _bf_sum(x):
    lane = lax.iota(jnp.int32, _L)
    for sh in (1, 2, 4, 8):
        x = x + _shuffle(x, lane ^ sh)
    return x


def _bf_max(x):
    lane = lax.iota(jnp.int32, _L)
    for sh in (1, 2, 4, 8):
        x = jnp.maximum(x, _shuffle(x, lane ^ sh))
    return x


def _make_edge_kernel(d, ew):
    mesh = plsc.VectorSubcoreMesh(core_axis_name="c", subcore_axis_name="s")
    grp = ew // _L

    npair = grp // 2
    tail = grp - npair * 2

    @functools.partial(
        pl.kernel,
        mesh=mesh,
        out_type=jax.ShapeDtypeStruct((ew * _NW,), jnp.float32),
        scratch_types=[
            pltpu.VMEM((ew,), jnp.int32),
            pltpu.VMEM((ew,), jnp.int32),
            pltpu.VMEM((_L, d), jnp.float32),
            pltpu.VMEM((_L, d), jnp.float32),
            pltpu.VMEM((_L, d), jnp.float32),
            pltpu.VMEM((_L, d), jnp.float32),
            pltpu.VMEM((ew,), jnp.float32),
            pltpu.VMEM((d,), jnp.float32),
            pltpu.SemaphoreType.DMA,
            pltpu.SemaphoreType.DMA,
            pltpu.SemaphoreType.DMA,
            pltpu.SemaphoreType.DMA,
        ],
        compiler_params=pltpu.CompilerParams(needs_layout_passes=False),
    )
    def edge_kernel(h_hbm, src_hbm, dst_hbm, avec_hbm, v_hbm,
                    srcv, dstv, hs0, hd0, hs1, hd1, vout, abuf,
                    sem1a, sem2a, sem1b, sem2b):
        wid = lax.axis_index("s") * _NC + lax.axis_index("c")
        base = wid * ew
        pltpu.sync_copy(src_hbm.at[pl.ds(base, ew)], srcv)
        pltpu.sync_copy(dst_hbm.at[pl.ds(base, ew)], dstv)
        pltpu.sync_copy(avec_hbm, abuf)
        lane = lax.iota(jnp.int32, _L)

        def compute(hs, hd, cs):
            vvec = jnp.zeros((_L,), jnp.float32)
            for e_i in range(_L):
                t = jnp.zeros((_L,), jnp.float32)
                for q in range(d // _L):
                    hsq = hs[e_i, pl.ds(q * _L, _L)]
                    hdq = hd[e_i, pl.ds(q * _L, _L)]
                    t = t + abuf[pl.ds(q * _L, _L)] * jnp.abs(hsq - hdq)
                vvec = vvec + jnp.where(lane == e_i, _bf_sum(t), 0.0)
            vout[pl.ds(cs, _L)] = jnp.maximum(vvec, 0.0)

        def pair(p, carry):
            cs0 = (2 * p) * _L
            cs1 = cs0 + _L
            cp1a = pltpu.async_copy(
                h_hbm.at[srcv.at[pl.ds(cs0, _L)]], hs0, sem1a)
            cp2a = pltpu.async_copy(
                h_hbm.at[dstv.at[pl.ds(cs0, _L)]], hd0, sem2a)
            cp1b = pltpu.async_copy(
                h_hbm.at[srcv.at[pl.ds(cs1, _L)]], hs1, sem1b)
            cp2b = pltpu.async_copy(
                h_hbm.at[dstv.at[pl.ds(cs1, _L)]], hd1, sem2b)
            cp1a.wait()
            cp2a.wait()
            compute(hs0, hd0, cs0)
            cp1b.wait()
            cp2b.wait()
            compute(hs1, hd1, cs1)
            return carry

        lax.fori_loop(0, npair, pair, 0)
        if tail:
            cs = (grp - 1) * _L
            cp1 = pltpu.async_copy(
                h_hbm.at[srcv.at[pl.ds(cs, _L)]], hs0, sem1a)
            cp2 = pltpu.async_copy(
                h_hbm.at[dstv.at[pl.ds(cs, _L)]], hd0, sem2a)
            cp1.wait()
            cp2.wait()
            compute(hs0, hd0, cs)
        pltpu.sync_copy(vout, v_hbm.at[pl.ds(base, ew)])

    return edge_kernel


_CAP = 64
_CAPC = _CAP // _L
_SCAP = 12288


def _make_row_kernel(n, rw, rw_pad):
    mesh = plsc.VectorSubcoreMesh(core_axis_name="c", subcore_axis_name="s")
    nq = n // _L
    nf = jnp.float32(n)

    @functools.partial(
        pl.kernel,
        mesh=mesh,
        out_type=jax.ShapeDtypeStruct((n, n), jnp.float32),
        scratch_types=[
            pltpu.VMEM((n,), jnp.float32),
            pltpu.VMEM((n,), jnp.float32),
            pltpu.VMEM((n,), jnp.float32),
            pltpu.VMEM((_CAP,), jnp.int32),
            pltpu.VMEM((_CAP,), jnp.float32),
            pltpu.VMEM((rw_pad,), jnp.int32),
            pltpu.VMEM((_SCAP,), jnp.int32),
            pltpu.VMEM((_SCAP,), jnp.float32),
            pltpu.SemaphoreType.DMA,
            pltpu.SemaphoreType.DMA,
            pltpu.SemaphoreType.DMA,
        ],
        compiler_params=pltpu.CompilerParams(needs_layout_passes=False),
    )
    def row_kernel(dst_hbm, v_hbm, rstart_hbm, out_hbm,
                   abuf, cbuf, outbuf, dstrow, vrow, rsv, dstb, vb,
                   semo, semb1, semb2):
        wid = lax.axis_index("s") * _NC + lax.axis_index("c")
        r0 = wid * rw
        nrows = jnp.clip(n - r0, 0, rw)
        pltpu.sync_copy(rstart_hbm.at[pl.ds(r0, rw_pad)], rsv)
        lane = lax.iota(jnp.int32, _L)
        zero16 = jnp.zeros((_L,), jnp.float32)
        ones16 = jnp.ones((_L,), jnp.float32)

        ebase = rsv[pl.ds(0, _L)][0]
        eend = rsv[pl.ds(nrows, _L)][0]
        eb0 = ebase // _L * _L
        bulk = ((eend + _L - 1) // _L * _L - eb0 + _CAP) <= _SCAP
        cpb1 = pltpu.async_copy(dst_hbm.at[pl.ds(eb0, _SCAP)], dstb, semb1)
        cpb2 = pltpu.async_copy(v_hbm.at[pl.ds(eb0, _SCAP)], vb, semb2)

        for q in range(nq):
            abuf[pl.ds(q * _L, _L)] = zero16
            cbuf[pl.ds(q * _L, _L)] = zero16
        cpb1.wait()
        cpb2.wait()

        def per_row(i, carry):
            row = r0 + i

            if True:
                rs = rsv[pl.ds(i, _L)][0]
                re = rsv[pl.ds(i + 1, _L)][0]
                c0 = rs // _L
                nch = (re + _L - 1) // _L - c0
                e0 = c0 * _L
                nseg = (nch + _CAPC - 1) // _CAPC

                def seg_bounds(s):
                    segbase = e0 + s * _CAP
                    jn = jnp.minimum(_CAPC, nch - s * _CAPC)
                    return segbase, jn

                def load_seg(segbase, with_v):
                    @pl.when(jnp.logical_not(bulk))
                    def _():
                        pltpu.sync_copy(
                            dst_hbm.at[pl.ds(segbase, _CAP)], dstrow)
                        if with_v:
                            pltpu.sync_copy(
                                v_hbm.at[pl.ds(segbase, _CAP)], vrow)

                def chunk_ctx(segbase, j):
                    ob = jnp.minimum(segbase - eb0 + j * _L, _SCAP - _L)
                    dstc = jnp.where(bulk, dstb[pl.ds(ob, _L)],
                                     dstrow[pl.ds(j * _L, _L)])
                    g = segbase + j * _L + lane
                    msk = (g >= rs) & (g < re)
                    return dstc, msk, ob

                def p1_seg(s, c):
                    segbase, jn = seg_bounds(s)
                    load_seg(segbase, True)

                    def p1_j(j, cc):
                        dstc, msk, ob = chunk_ctx(segbase, j)
                        vc = jnp.where(bulk, vb[pl.ds(ob, _L)],
                                       vrow[pl.ds(j * _L, _L)])
                        plsc.addupdate_scatter(abuf, [dstc], vc, mask=msk)
                        plsc.addupdate_scatter(cbuf, [dstc], ones16, mask=msk)
                        return cc
                    lax.fori_loop(0, jn, p1_j, 0)
                    return c
                lax.fori_loop(0, nseg, p1_seg, 0)

                def p2_seg(s, mv):
                    segbase, jn = seg_bounds(s)

                    @pl.when(nseg > 1)
                    def _():
                        load_seg(segbase, False)

                    def p2_j(j, mvj):
                        dstc, msk, _ = chunk_ctx(segbase, j)
                        aj = plsc.load_gather(abuf, [dstc])
                        return jnp.maximum(mvj, jnp.where(msk, aj, 0.0))
                    return lax.fori_loop(0, jn, p2_j, mv)
                mvec = lax.fori_loop(0, nseg, p2_seg, zero16)
                msp = _bf_max(mvec)
                em = jnp.exp(-msp)

                def p3_seg(s, zv):
                    segbase, jn = seg_bounds(s)

                    @pl.when(nseg > 1)
                    def _():
                        load_seg(segbase, False)

                    def p3_j(j, zvj):
                        dstc, msk, _ = chunk_ctx(segbase, j)
                        aj = plsc.load_gather(abuf, [dstc])
                        cj = plsc.load_gather(cbuf, [dstc])
                        t = (jnp.exp(aj - msp) - em) / cj
                        return zvj + jnp.where(msk, t, 0.0)
                    return lax.fori_loop(0, jn, p3_j, zv)
                zacc = lax.fori_loop(0, nseg, p3_seg, zero16)
                zsp = nf * em + _bf_sum(zacc)
                rzsp = 1.0 / zsp
                bsp = em * rzsp

                @pl.when(i > 0)
                def _():
                    pltpu.make_async_copy(
                        outbuf, out_hbm.at[row - 1], semo).wait()

                for q in range(nq):
                    outbuf[pl.ds(q * _L, _L)] = bsp

                def p4_seg(s, c):
                    segbase, jn = seg_bounds(s)

                    @pl.when(nseg > 1)
                    def _():
                        load_seg(segbase, False)

                    def p4_j(j, cc):
                        dstc, msk, _ = chunk_ctx(segbase, j)
                        aj = plsc.load_gather(abuf, [dstc])
                        cj = plsc.load_gather(cbuf, [dstc])
                        w = jnp.exp(aj - msp) * rzsp + cj
                        plsc.store_scatter(outbuf, [dstc], w, mask=msk)
                        return cc
                    lax.fori_loop(0, jn, p4_j, 0)
                    return c
                lax.fori_loop(0, nseg, p4_seg, 0)

                pltpu.async_copy(outbuf, out_hbm.at[row], semo)

                def p5_seg(s, c):
                    segbase, jn = seg_bounds(s)

                    @pl.when(nseg > 1)
                    def _():
                        load_seg(segbase, False)

                    def p5_j(j, cc):
                        dstc, msk, _ = chunk_ctx(segbase, j)
                        plsc.store_scatter(abuf, [dstc], zero16, mask=msk)
                        plsc.store_scatter(cbuf, [dstc], zero16, mask=msk)
                        return cc
                    lax.fori_loop(0, jn, p5_j, 0)
                    return c
                lax.fori_loop(0, nseg, p5_seg, 0)

            return carry

        lax.fori_loop(0, nrows, per_row, 0)

        @pl.when(nrows > 0)
        def _():
            pltpu.make_async_copy(
                outbuf, out_hbm.at[r0 + nrows - 1], semo).wait()

    return row_kernel


def kernel(inputs, edge, W, a):
    n, d_in = inputs.shape
    d_out = W.shape[1]
    e = edge.shape[1]

    src = edge[0].astype(jnp.int32)
    dst = edge[1].astype(jnp.int32)

    ew = _ceil_to(e, _NW * _L) // _NW
    e_pad = ew * _NW
    pad = e_pad - e
    if pad:
        src = jnp.concatenate([src, jnp.full((pad,), n, jnp.int32)])
        dst = jnp.concatenate([dst, jnp.zeros((pad,), jnp.int32)])
    src_g = jnp.minimum(src, n - 1)

    rw = _ceil_to(_ceil_to(n, _NW) // _NW, 8)
    rw_pad = rw + _L
    gl = _NW * rw + _L

    bm = 2000
    h = pl.pallas_call(
        _matmul_block,
        grid=(n // bm,),
        in_specs=[pl.BlockSpec((bm, d_in), lambda i: (i, 0)),
                  pl.BlockSpec((d_in, d_out), lambda i: (0, 0))],
        out_specs=pl.BlockSpec((bm, d_out), lambda i: (i, 0)),
        out_shape=jax.ShapeDtypeStruct((n, d_out), jnp.float32),
    )(inputs, W)

    avec = a.astype(jnp.float32).reshape(d_out)
    v = _make_edge_kernel(d_out, ew)(h, src_g, dst, avec)

    iot = jnp.arange(e_pad, dtype=jnp.int32)
    src_s, dst_s, order = lax.sort([src, dst, iot],
                                   num_keys=1, is_stable=False)
    row_start = jnp.searchsorted(
        src_s, jnp.minimum(jnp.arange(gl), n)).astype(jnp.int32)
    v_s = v[order]

    dst3 = jnp.concatenate([dst_s, jnp.zeros((_SCAP,), jnp.int32)])
    v3 = jnp.concatenate([v_s, jnp.zeros((_SCAP,), jnp.float32)])
    sgraph = _make_row_kernel(n, rw, rw_pad)(dst3, v3, row_start)

    return (h, sgraph)
